# Initial kernel scaffold; baseline (speedup 1.0000x reference)
import jax
import jax.numpy as jnp
from jax.experimental import pallas as pl


def kernel(query_seq, matching_set, topk):
    raise NotImplementedError("write your pallas kernel here")


# speedup vs baseline: 1.9825x; 1.9825x over previous
import functools

import jax
import jax.numpy as jnp
from jax import lax
from jax.experimental import pallas as pl
from jax.experimental.pallas import tpu as pltpu
from jax.experimental.pallas import tpu_sc as plsc

Q = 1024
N = 16384
D = 1024
K = 4

QB = 256
NB = 2048

_INF = float("inf")
_BIGI = 2**30

_NC = 2
_NS = 16
_NW = _NC * _NS
_QPW = Q // _NW
_GRP = 8


def _topk_body(nq_ref, nm_ref, q_ref, m_ref, out_ref, bv_ref, bi_ref):
    j = pl.program_id(1)

    @pl.when(j == 0)
    def _init():
        bv_ref[...] = jnp.full((QB, K), _INF, jnp.float32)
        bi_ref[...] = jnp.full((QB, K), _BIGI, jnp.int32)

    dot = lax.dot_general(
        q_ref[...], m_ref[...],
        (((1,), (1,)), ((), ())),
        preferred_element_type=jnp.float32,
    )

    nq = nq_ref[...]
    nm = nm_ref[...]
    nq2 = nq * nq
    nm2 = nm * nm
    cdist_sq = nq2 + nm2 - 2.0 * dot
    dotprod = (-cdist_sq + nq2 + nm2) / 2.0
    dists = 1.0 - dotprod / (nq * nm)

    cols = lax.broadcasted_iota(jnp.int32, (QB, NB), 1) + j * NB

    cand_v, cand_i = [], []
    d = dists
    for _ in range(K):
        vmin = jnp.min(d, axis=1, keepdims=True)
        imin = jnp.min(jnp.where(d == vmin, cols, _BIGI), axis=1,
                       keepdims=True)
        cand_v.append(vmin)
        cand_i.append(imin)
        d = jnp.where(cols == imin, _INF, d)

    vals = [bv_ref[:, t:t + 1] for t in range(K)] + cand_v
    idxs = [bi_ref[:, t:t + 1] for t in range(K)] + cand_i
    for t in range(K):
        vmin = vals[0]
        for v in vals[1:]:
            vmin = jnp.minimum(vmin, v)
        taken = jnp.zeros_like(vmin, dtype=jnp.bool_)
        sel_i = jnp.full_like(idxs[0], _BIGI)
        for p in range(len(vals)):
            hit = (vals[p] == vmin) & jnp.logical_not(taken)
            sel_i = jnp.where(hit, idxs[p], sel_i)
            vals[p] = jnp.where(hit, _INF, vals[p])
            taken = taken | hit
        bv_ref[:, t:t + 1] = vmin
        bi_ref[:, t:t + 1] = sel_i
        out_ref[:, t:t + 1] = sel_i


_topk_call = pl.pallas_call(
    _topk_body,
    grid=(Q // QB, N // NB),
    in_specs=[
        pl.BlockSpec((QB, 1), lambda i, j: (i, 0)),
        pl.BlockSpec((1, NB), lambda i, j: (0, j)),
        pl.BlockSpec((QB, D), lambda i, j: (i, 0)),
        pl.BlockSpec((NB, D), lambda i, j: (j, 0)),
    ],
    out_specs=pl.BlockSpec((QB, K), lambda i, j: (i, 0)),
    out_shape=jax.ShapeDtypeStruct((Q, K), jnp.int32),
    scratch_shapes=[
        pltpu.VMEM((QB, K), jnp.float32),
        pltpu.VMEM((QB, K), jnp.int32),
    ],
    compiler_params=pltpu.CompilerParams(
        dimension_semantics=("parallel", "arbitrary"),
    ),
)


def _gather_mean_body(ms_hbm, idx_hbm, out_hbm, idx_v, rows_v, out_v, sem):
    wid = lax.axis_index("s") * _NC + lax.axis_index("c")
    qbase = wid * _QPW
    for g in range(_QPW // _GRP):
        q0 = qbase + g * _GRP
        pltpu.sync_copy(idx_hbm.at[pl.ds(q0 * K, _GRP * K)], idx_v)
        pltpu.async_copy(ms_hbm.at[idx_v], rows_v, sem).wait()

        def _chunk(c, _):
            sl = pl.ds(c * 16, 16)
            for q in range(_GRP):
                s = (rows_v[K * q, sl] + rows_v[K * q + 1, sl]
                     + rows_v[K * q + 2, sl] + rows_v[K * q + 3, sl])
                out_v[q, sl] = s * 0.25
            return _

        lax.fori_loop(0, D // 16, _chunk, None)
        pltpu.sync_copy(out_v, out_hbm.at[pl.ds(q0, _GRP)])


@functools.cache
def _gather_mean_call():
    return pl.kernel(
        _gather_mean_body,
        mesh=plsc.VectorSubcoreMesh(core_axis_name="c", subcore_axis_name="s",
                                    num_cores=_NC, num_subcores=_NS),
        out_type=jax.ShapeDtypeStruct((Q, D), jnp.float32),
        scratch_types=[
            pltpu.VMEM((_GRP * K,), jnp.int32),
            pltpu.VMEM((_GRP * K, D), jnp.float32),
            pltpu.VMEM((_GRP, D), jnp.float32),
            pltpu.SemaphoreType.DMA,
        ],
    )


def kernel(query_seq, matching_set, topk):
    del topk
    nq = jnp.linalg.norm(query_seq, ord=2, axis=-1)
    nm = jnp.linalg.norm(matching_set, ord=2, axis=-1)
    idx = _topk_call(nq.reshape(Q, 1), nm.reshape(1, N),
                     query_seq, matching_set)
    return _gather_mean_call()(matching_set, idx.reshape(Q * K))


# speedup vs baseline: 2.9472x; 1.4866x over previous
import functools

import jax
import jax.numpy as jnp
from jax import lax
from jax.experimental import pallas as pl
from jax.experimental.pallas import tpu as pltpu
from jax.experimental.pallas import tpu_sc as plsc

Q = 1024
N = 16384
D = 1024
K = 4

QB = 256
NB = 2048

_INF = float("inf")
_BIGI = 2**30

_NC = 2
_NS = 16
_NW = _NC * _NS
_QPW = Q // _NW
_GRP = 8


_NJ = N // NB


def _topk_body(nq_ref, nm_ref, q_ref, m_ref, out_ref, cv_ref, ci_ref):
    j = pl.program_id(1)

    dot = lax.dot_general(
        q_ref[...], m_ref[...],
        (((1,), (1,)), ((), ())),
        preferred_element_type=jnp.float32,
    )

    nq = nq_ref[...]
    nm = nm_ref[...]
    nq2 = nq * nq
    nm2 = nm * nm
    cdist_sq = (nq2 + nm2) - 2.0 * dot
    dotprod = (nq2 - cdist_sq) + nm2
    dists = 1.0 - dotprod / (2.0 * (nq * nm))

    cols = lax.broadcasted_iota(jnp.int32, (QB, NB), 1) + j * NB

    cand_v, cand_i = [], []
    d = dists
    for t in range(K):
        vmin = jnp.min(d, axis=1, keepdims=True)
        imin = jnp.min(jnp.where(d == vmin, cols, _BIGI), axis=1,
                       keepdims=True)
        cand_v.append(vmin)
        cand_i.append(imin)
        if t + 1 < K:
            d = jnp.where(cols == imin, _INF, d)

    for jj in range(_NJ):
        @pl.when(j == jj)
        def _stage(jj=jj):
            for t in range(K):
                s = jj * K + t
                cv_ref[:, s:s + 1] = cand_v[t]
                ci_ref[:, s:s + 1] = cand_i[t]

    @pl.when(j == _NJ - 1)
    def _final():
        v = cv_ref[...]
        idx = ci_ref[...]
        pos = lax.broadcasted_iota(jnp.int32, (QB, _NJ * K), 1)
        vv = v
        for t in range(K):
            vmin = jnp.min(vv, axis=1, keepdims=True)
            p = jnp.min(jnp.where(vv == vmin, pos, _BIGI), axis=1,
                        keepdims=True)
            hit = pos == p
            sel = jnp.min(jnp.where(hit, idx, _BIGI), axis=1, keepdims=True)
            out_ref[:, t:t + 1] = sel
            if t + 1 < K:
                vv = jnp.where(hit, _INF, vv)


_topk_call = pl.pallas_call(
    _topk_body,
    grid=(Q // QB, N // NB),
    in_specs=[
        pl.BlockSpec((QB, 1), lambda i, j: (i, 0)),
        pl.BlockSpec((1, NB), lambda i, j: (0, j)),
        pl.BlockSpec((QB, D), lambda i, j: (i, 0)),
        pl.BlockSpec((NB, D), lambda i, j: (j, 0)),
    ],
    out_specs=pl.BlockSpec((QB, K), lambda i, j: (i, 0)),
    out_shape=jax.ShapeDtypeStruct((Q, K), jnp.int32),
    scratch_shapes=[
        pltpu.VMEM((QB, _NJ * K), jnp.float32),
        pltpu.VMEM((QB, _NJ * K), jnp.int32),
    ],
    compiler_params=pltpu.CompilerParams(
        dimension_semantics=("parallel", "arbitrary"),
    ),
)


def _gather_mean_body(ms_hbm, idx_hbm, out_hbm, idx_v, rows_v, out_v, sem):
    wid = lax.axis_index("s") * _NC + lax.axis_index("c")
    qbase = wid * _QPW
    for g in range(_QPW // _GRP):
        q0 = qbase + g * _GRP
        pltpu.sync_copy(idx_hbm.at[pl.ds(q0 * K, _GRP * K)], idx_v)
        pltpu.async_copy(ms_hbm.at[idx_v], rows_v, sem).wait()

        def _chunk(c, _):
            sl = pl.ds(c * 16, 16)
            for q in range(_GRP):
                s = (rows_v[K * q, sl] + rows_v[K * q + 1, sl]
                     + rows_v[K * q + 2, sl] + rows_v[K * q + 3, sl])
                out_v[q, sl] = s * 0.25
            return _

        lax.fori_loop(0, D // 16, _chunk, None)
        pltpu.sync_copy(out_v, out_hbm.at[pl.ds(q0, _GRP)])


@functools.cache
def _gather_mean_call():
    return pl.kernel(
        _gather_mean_body,
        mesh=plsc.VectorSubcoreMesh(core_axis_name="c", subcore_axis_name="s",
                                    num_cores=_NC, num_subcores=_NS),
        out_type=jax.ShapeDtypeStruct((Q, D), jnp.float32),
        scratch_types=[
            pltpu.VMEM((_GRP * K,), jnp.int32),
            pltpu.VMEM((_GRP * K, D), jnp.float32),
            pltpu.VMEM((_GRP, D), jnp.float32),
            pltpu.SemaphoreType.DMA,
        ],
    )


def kernel(query_seq, matching_set, topk):
    del topk
    nq = jnp.linalg.norm(query_seq, ord=2, axis=-1)
    nm = jnp.linalg.norm(matching_set, ord=2, axis=-1)
    idx = _topk_call(nq.reshape(Q, 1), nm.reshape(1, N),
                     query_seq, matching_set)
    return _gather_mean_call()(matching_set, idx.reshape(Q * K))
